# baseline (speedup 1.0000x reference)
import jax
import jax.numpy as jnp
import numpy as np
from jax.experimental import pallas as pl

_B, _K, _L = 16, 100, 200
_TIME = 128
_FEAT = 16
_C = _TIME + _FEAT


def _body(tp_ref, div2_ref, off_ref, ett_ref, out_ref):
    pos = tp_ref[0, 0, :]
    angle = div2_ref[...] * pos[None, :] + off_ref[...]
    pe = jnp.sin(angle)
    out_ref[0, 0:_TIME, :, :] = jnp.broadcast_to(
        pe[:, None, :], (_TIME, _K, _L)
    )
    out_ref[0, _TIME:_C, :, :] = jnp.broadcast_to(
        ett_ref[...][:, :, None], (_FEAT, _K, _L)
    )


def kernel(observed_tp, observed_mask, embed_table):
    del observed_mask
    half = jnp.power(
        10000.0, -jnp.arange(0, _TIME, 2, dtype=jnp.float32) / _TIME
    )
    div2 = jnp.repeat(half, 2).reshape(_TIME, 1)
    off = jnp.tile(jnp.array([0.0, np.pi / 2], jnp.float32), _TIME // 2)
    off = off.reshape(_TIME, 1)
    ett = embed_table.T
    tp3 = observed_tp.reshape(_B, 1, _L)

    out = pl.pallas_call(
        _body,
        grid=(_B,),
        in_specs=[
            pl.BlockSpec((1, 1, _L), lambda b: (b, 0, 0)),
            pl.BlockSpec((_TIME, 1), lambda b: (0, 0)),
            pl.BlockSpec((_TIME, 1), lambda b: (0, 0)),
            pl.BlockSpec((_FEAT, _K), lambda b: (0, 0)),
        ],
        out_specs=pl.BlockSpec((1, _C, _K, _L), lambda b: (b, 0, 0, 0)),
        out_shape=jax.ShapeDtypeStruct((_B, _C, _K, _L), jnp.float32),
    )(tp3, div2, off, ett)
    return out
